# baseline (speedup 1.0000x reference)
import functools

import jax
import jax.numpy as jnp
from jax import lax
from jax.experimental import pallas as pl
from jax.experimental.pallas import tpu as pltpu
from jax.experimental.pallas import tpu_sc as plsc

_B, _N = 8, 65536
_PC, _AUG, _H = 3, 61, 128
_IN = _PC + _AUG
_PERCENT = 0.1
_TN = 2048


def _mlp_body(f_ref, xyz_ref, w1_ref, b1_ref, w2_ref, b2_ref, w3_ref, b3_ref,
              wp_ref, bp_ref, nrm_ref, wd1_ref, bd1_ref, wd2_ref, bd2_ref,
              wd3_ref, bd3_ref, probs_ref, cand_ref):
    fb = f_ref[...]
    xyzb = xyz_ref[...]
    x = jnp.concatenate([fb, xyzb], axis=-1)
    h = lax.dot(x, w1_ref[...]) + b1_ref[...]
    h = jnp.where(h >= 0, h, 0.01 * h)
    h = lax.dot(h, w2_ref[...]) + b2_ref[...]
    h = jnp.where(h >= 0, h, 0.01 * h)
    h = lax.dot(h, w3_ref[...]) + b3_ref[...]
    h = jnp.maximum(h, 0.0)
    p = (lax.dot(h, wp_ref[...]) + bp_ref[...]) / nrm_ref[0, 0]
    probs_ref[...] = p
    y = jax.nn.sigmoid(p)
    px = fb * y
    d = lax.dot(px, wd1_ref[...]) + bd1_ref[...]
    d = jnp.maximum(d, 0.0)
    d = lax.dot(d, wd2_ref[...]) + bd2_ref[...]
    d = jnp.maximum(d, 0.0)
    d = lax.dot(d, wd3_ref[...]) + bd3_ref[...]
    cand = d + jnp.concatenate(
        [xyzb, jnp.zeros((xyzb.shape[0], 125), jnp.float32)], axis=-1)
    cand_ref[...] = cand


def _full(shape):
    return pl.BlockSpec(shape, lambda i: tuple(0 for _ in shape))


_NW = 32
_G_CHUNK = 928
_G_NCHUNK = 16
_G_PER_W = _G_CHUNK * _G_NCHUNK
_G_TOTAL = _G_PER_W * _NW


def _sc_gather_body(cand_hbm, idx_hbm, out_hbm, idx_v, rows_v, sem):
    wid = lax.axis_index("s") * 2 + lax.axis_index("c")
    base = wid * _G_PER_W
    for c in range(_G_NCHUNK):
        off = base + c * _G_CHUNK
        pltpu.sync_copy(idx_hbm.at[pl.ds(off, _G_CHUNK)], idx_v)
        pltpu.async_copy(cand_hbm.at[idx_v], rows_v, sem).wait()
        pltpu.sync_copy(rows_v, out_hbm.at[pl.ds(off, _G_CHUNK)])


_sc_gather = pl.kernel(
    _sc_gather_body,
    out_type=jax.ShapeDtypeStruct((_G_TOTAL, 128), jnp.float32),
    mesh=plsc.VectorSubcoreMesh(core_axis_name="c", subcore_axis_name="s"),
    scratch_types=[
        pltpu.VMEM((_G_CHUNK,), jnp.int32),
        pltpu.VMEM((_G_CHUNK, 128), jnp.float32),
        pltpu.SemaphoreType.DMA,
    ],
)


@jax.jit
def kernel(xyz, f, W1, b1, W2, b2, W3, b3, Wp, bp, Wd1, bd1, Wd2, bd2, Wd3, bd3):
    BN = _B * _N
    num_out = int(_N * _PERCENT)
    f2 = f.reshape(BN, _IN)
    xyz2 = xyz.reshape(BN, _PC)
    nrm = jnp.linalg.norm(Wp).reshape(1, 1)
    wd3t = jnp.zeros((_IN // 4, 128), jnp.float32).at[:, :_PC].set(Wd3.T)
    bd3p = jnp.zeros((1, 128), jnp.float32).at[0, :_PC].set(bd3)

    grid = (BN // _TN,)
    probs2, cand = pl.pallas_call(
        _mlp_body,
        grid=grid,
        in_specs=[
            pl.BlockSpec((_TN, _IN), lambda i: (i, 0)),
            pl.BlockSpec((_TN, _PC), lambda i: (i, 0)),
            _full((_IN + _PC, _H)),
            _full((1, _H)),
            _full((_H, _H)),
            _full((1, _H)),
            _full((_H, 32)),
            _full((1, 32)),
            _full((32, 1)),
            _full((1, 1)),
            _full((1, 1)),
            _full((_IN, _IN // 2)),
            _full((1, _IN // 2)),
            _full((_IN // 2, _IN // 4)),
            _full((1, _IN // 4)),
            _full((_IN // 4, 128)),
            _full((1, 128)),
        ],
        out_specs=[
            pl.BlockSpec((_TN, 1), lambda i: (i, 0)),
            pl.BlockSpec((_TN, 128), lambda i: (i, 0)),
        ],
        out_shape=[
            jax.ShapeDtypeStruct((BN, 1), jnp.float32),
            jax.ShapeDtypeStruct((BN, 128), jnp.float32),
        ],
        compiler_params=pltpu.CompilerParams(
            dimension_semantics=("arbitrary",),
        ),
    )(f2, xyz2, W1.T, b1.reshape(1, _H), W2.T, b2.reshape(1, _H), W3.T,
      b3.reshape(1, 32), Wp.T, bp.reshape(1, 1), nrm, Wd1.T,
      bd1.reshape(1, _IN // 2), Wd2.T, bd2.reshape(1, _IN // 4), wd3t, bd3p)

    probs = probs2.reshape(_B, _N)
    prob_idx = jnp.argsort(-probs, axis=-1)
    clean_idx = prob_idx[:, num_out:]
    nc = _N - num_out
    gidx = (jnp.arange(_B, dtype=jnp.int32)[:, None] * _N + clean_idx).reshape(-1)
    gidx = jnp.pad(gidx, (0, _G_TOTAL - _B * nc))
    gathered = _sc_gather(cand, gidx)
    est_xyz = gathered[: _B * nc, :_PC].reshape(_B, nc, _PC)
    return (prob_idx, est_xyz)


# speedup vs baseline: 1.0106x; 1.0106x over previous
import functools

import jax
import jax.numpy as jnp
from jax import lax
from jax.experimental import pallas as pl
from jax.experimental.pallas import tpu as pltpu
from jax.experimental.pallas import tpu_sc as plsc

_B, _N = 8, 65536
_PC, _AUG, _H = 3, 61, 128
_IN = _PC + _AUG
_PERCENT = 0.1
_TN = 2048


def _mlp_body(f_ref, xyz_ref, w1_ref, b1_ref, w2_ref, b2_ref, w3_ref, b3_ref,
              wp_ref, bp_ref, nrm_ref, wd1_ref, bd1_ref, wd2_ref, bd2_ref,
              wd3_ref, bd3_ref, cand_ref):
    fb = f_ref[...]
    xyzb = xyz_ref[...]
    x = jnp.concatenate([fb, xyzb], axis=-1)
    h = lax.dot(x, w1_ref[...]) + b1_ref[...]
    h = jnp.where(h >= 0, h, 0.01 * h)
    h = lax.dot(h, w2_ref[...]) + b2_ref[...]
    h = jnp.where(h >= 0, h, 0.01 * h)
    h = lax.dot(h, w3_ref[...]) + b3_ref[...]
    h = jnp.maximum(h, 0.0)
    p = (lax.dot(h, wp_ref[...]) + bp_ref[...]) / nrm_ref[0, 0]
    y = jax.nn.sigmoid(p)
    px = fb * y
    d = lax.dot(px, wd1_ref[...]) + bd1_ref[...]
    d = jnp.maximum(d, 0.0)
    d = lax.dot(d, wd2_ref[...]) + bd2_ref[...]
    d = jnp.maximum(d, 0.0)
    d = lax.dot(d, wd3_ref[...]) + bd3_ref[...]
    cand = d + jnp.concatenate(
        [xyzb, p, jnp.zeros((xyzb.shape[0], 124), jnp.float32)], axis=-1)
    cand_ref[...] = cand


def _full(shape):
    return pl.BlockSpec(shape, lambda i: tuple(0 for _ in shape))


_NW = 32
_G_CHUNK = 928
_G_NCHUNK = 16
_G_PER_W = _G_CHUNK * _G_NCHUNK
_G_TOTAL = _G_PER_W * _NW


def _sc_gather_body(cand_hbm, idx_hbm, out_hbm, idx_v, rows_v, sem):
    wid = lax.axis_index("s") * 2 + lax.axis_index("c")
    base = wid * _G_PER_W
    for c in range(_G_NCHUNK):
        off = base + c * _G_CHUNK
        pltpu.sync_copy(idx_hbm.at[pl.ds(off, _G_CHUNK)], idx_v)
        pltpu.async_copy(cand_hbm.at[idx_v], rows_v, sem).wait()
        pltpu.sync_copy(rows_v, out_hbm.at[pl.ds(off, _G_CHUNK)])


_sc_gather = pl.kernel(
    _sc_gather_body,
    out_type=jax.ShapeDtypeStruct((_G_TOTAL, 128), jnp.float32),
    mesh=plsc.VectorSubcoreMesh(core_axis_name="c", subcore_axis_name="s"),
    scratch_types=[
        pltpu.VMEM((_G_CHUNK,), jnp.int32),
        pltpu.VMEM((_G_CHUNK, 128), jnp.float32),
        pltpu.SemaphoreType.DMA,
    ],
)


@jax.jit
def kernel(xyz, f, W1, b1, W2, b2, W3, b3, Wp, bp, Wd1, bd1, Wd2, bd2, Wd3, bd3):
    BN = _B * _N
    num_out = int(_N * _PERCENT)
    f2 = f.reshape(BN, _IN)
    xyz2 = xyz.reshape(BN, _PC)
    nrm = jnp.linalg.norm(Wp).reshape(1, 1)
    wd3t = jnp.zeros((_IN // 4, 128), jnp.float32).at[:, :_PC].set(Wd3.T)
    bd3p = jnp.zeros((1, 128), jnp.float32).at[0, :_PC].set(bd3)

    grid = (BN // _TN,)
    cand = pl.pallas_call(
        _mlp_body,
        grid=grid,
        in_specs=[
            pl.BlockSpec((_TN, _IN), lambda i: (i, 0)),
            pl.BlockSpec((_TN, _PC), lambda i: (i, 0)),
            _full((_IN + _PC, _H)),
            _full((1, _H)),
            _full((_H, _H)),
            _full((1, _H)),
            _full((_H, 32)),
            _full((1, 32)),
            _full((32, 1)),
            _full((1, 1)),
            _full((1, 1)),
            _full((_IN, _IN // 2)),
            _full((1, _IN // 2)),
            _full((_IN // 2, _IN // 4)),
            _full((1, _IN // 4)),
            _full((_IN // 4, 128)),
            _full((1, 128)),
        ],
        out_specs=pl.BlockSpec((_TN, 128), lambda i: (i, 0)),
        out_shape=jax.ShapeDtypeStruct((BN, 128), jnp.float32),
        compiler_params=pltpu.CompilerParams(
            dimension_semantics=("arbitrary",),
        ),
    )(f2, xyz2, W1.T, b1.reshape(1, _H), W2.T, b2.reshape(1, _H), W3.T,
      b3.reshape(1, 32), Wp.T, bp.reshape(1, 1), nrm, Wd1.T,
      bd1.reshape(1, _IN // 2), Wd2.T, bd2.reshape(1, _IN // 4), wd3t, bd3p)

    probs = cand[:, 3].reshape(_B, _N)
    prob_idx = jnp.argsort(-probs, axis=-1)
    clean_idx = prob_idx[:, num_out:]
    nc = _N - num_out
    gidx = (jnp.arange(_B, dtype=jnp.int32)[:, None] * _N + clean_idx).reshape(-1)
    gidx = jnp.pad(gidx, (0, _G_TOTAL - _B * nc))
    gathered = _sc_gather(cand, gidx)
    est_xyz = gathered[: _B * nc, :_PC].reshape(_B, nc, _PC)
    return (prob_idx, est_xyz)


# speedup vs baseline: 1.0110x; 1.0004x over previous
import functools

import jax
import jax.numpy as jnp
from jax import lax
from jax.experimental import pallas as pl
from jax.experimental.pallas import tpu as pltpu
from jax.experimental.pallas import tpu_sc as plsc

_B, _N = 8, 65536
_PC, _AUG, _H = 3, 61, 128
_IN = _PC + _AUG
_PERCENT = 0.1
_TN = 2048


def _mlp_body(f_ref, xyz_ref, w1_ref, b1_ref, w2_ref, b2_ref, w3_ref, b3_ref,
              wp_ref, bp_ref, nrm_ref, wd1_ref, bd1_ref, wd2_ref, bd2_ref,
              wd3_ref, bd3_ref, cand_ref):
    fb = f_ref[...]
    xyzb = xyz_ref[...]
    x = jnp.concatenate([fb, xyzb], axis=-1)
    h = lax.dot(x, w1_ref[...]) + b1_ref[...]
    h = jnp.where(h >= 0, h, 0.01 * h)
    h = lax.dot(h, w2_ref[...]) + b2_ref[...]
    h = jnp.where(h >= 0, h, 0.01 * h)
    h = lax.dot(h, w3_ref[...]) + b3_ref[...]
    h = jnp.maximum(h, 0.0)
    p = (lax.dot(h, wp_ref[...]) + bp_ref[...]) / nrm_ref[0, 0]
    y = jax.nn.sigmoid(p)
    px = fb * y
    d = lax.dot(px, wd1_ref[...]) + bd1_ref[...]
    d = jnp.maximum(d, 0.0)
    d = lax.dot(d, wd2_ref[...]) + bd2_ref[...]
    d = jnp.maximum(d, 0.0)
    d = lax.dot(d, wd3_ref[...]) + bd3_ref[...]
    cand = d + jnp.concatenate(
        [xyzb, p, jnp.zeros((xyzb.shape[0], 124), jnp.float32)], axis=-1)
    cand_ref[...] = cand


def _full(shape):
    return pl.BlockSpec(shape, lambda i: tuple(0 for _ in shape))


_NW = 32
_G_CHUNK = 928
_G_NCHUNK = 16
_G_PER_W = _G_CHUNK * _G_NCHUNK
_G_TOTAL = _G_PER_W * _NW


def _sc_gather_body(cand_hbm, idx_hbm, out_hbm, idx_v, rows_v, sem):
    wid = lax.axis_index("s") * 2 + lax.axis_index("c")
    base = wid * _G_PER_W
    for c in range(_G_NCHUNK):
        off = base + c * _G_CHUNK
        pltpu.sync_copy(idx_hbm.at[pl.ds(off, _G_CHUNK)], idx_v)
        pltpu.async_copy(cand_hbm.at[idx_v], rows_v, sem).wait()
        pltpu.sync_copy(rows_v, out_hbm.at[pl.ds(off, _G_CHUNK)])


_sc_gather = pl.kernel(
    _sc_gather_body,
    out_type=jax.ShapeDtypeStruct((_G_TOTAL, 128), jnp.float32),
    mesh=plsc.VectorSubcoreMesh(core_axis_name="c", subcore_axis_name="s"),
    scratch_types=[
        pltpu.VMEM((_G_CHUNK,), jnp.int32),
        pltpu.VMEM((_G_CHUNK, 128), jnp.float32),
        pltpu.SemaphoreType.DMA,
    ],
    compiler_params=pltpu.CompilerParams(use_tc_tiling_on_sc=True),
)


@jax.jit
def kernel(xyz, f, W1, b1, W2, b2, W3, b3, Wp, bp, Wd1, bd1, Wd2, bd2, Wd3, bd3):
    BN = _B * _N
    num_out = int(_N * _PERCENT)
    f2 = f.reshape(BN, _IN)
    xyz2 = xyz.reshape(BN, _PC)
    nrm = jnp.linalg.norm(Wp).reshape(1, 1)
    wd3t = jnp.zeros((_IN // 4, 128), jnp.float32).at[:, :_PC].set(Wd3.T)
    bd3p = jnp.zeros((1, 128), jnp.float32).at[0, :_PC].set(bd3)

    grid = (BN // _TN,)
    cand = pl.pallas_call(
        _mlp_body,
        grid=grid,
        in_specs=[
            pl.BlockSpec((_TN, _IN), lambda i: (i, 0)),
            pl.BlockSpec((_TN, _PC), lambda i: (i, 0)),
            _full((_IN + _PC, _H)),
            _full((1, _H)),
            _full((_H, _H)),
            _full((1, _H)),
            _full((_H, 32)),
            _full((1, 32)),
            _full((32, 1)),
            _full((1, 1)),
            _full((1, 1)),
            _full((_IN, _IN // 2)),
            _full((1, _IN // 2)),
            _full((_IN // 2, _IN // 4)),
            _full((1, _IN // 4)),
            _full((_IN // 4, 128)),
            _full((1, 128)),
        ],
        out_specs=pl.BlockSpec((_TN, 128), lambda i: (i, 0)),
        out_shape=jax.ShapeDtypeStruct((BN, 128), jnp.float32),
        compiler_params=pltpu.CompilerParams(
            dimension_semantics=("arbitrary",),
        ),
    )(f2, xyz2, W1.T, b1.reshape(1, _H), W2.T, b2.reshape(1, _H), W3.T,
      b3.reshape(1, 32), Wp.T, bp.reshape(1, 1), nrm, Wd1.T,
      bd1.reshape(1, _IN // 2), Wd2.T, bd2.reshape(1, _IN // 4), wd3t, bd3p)

    probs = cand[:, 3].reshape(_B, _N)
    prob_idx = jnp.argsort(-probs, axis=-1)
    clean_idx = prob_idx[:, num_out:]
    nc = _N - num_out
    gidx = (jnp.arange(_B, dtype=jnp.int32)[:, None] * _N + clean_idx).reshape(-1)
    gidx = jnp.pad(gidx, (0, _G_TOTAL - _B * nc))
    gathered = _sc_gather(cand, gidx)
    est_xyz = gathered[: _B * nc, :_PC].reshape(_B, nc, _PC)
    return (prob_idx, est_xyz)


# speedup vs baseline: 1.8088x; 1.7890x over previous
import functools

import jax
import jax.numpy as jnp
from jax import lax
from jax.experimental import pallas as pl
from jax.experimental.pallas import tpu as pltpu
from jax.experimental.pallas import tpu_sc as plsc

_B, _N = 8, 65536
_PC, _AUG, _H = 3, 61, 128
_IN = _PC + _AUG
_PERCENT = 0.1
_TN = 2048


def _mlp_body(f_ref, xyz_ref, w1_ref, b1_ref, w2_ref, b2_ref, w3_ref, b3_ref,
              wp_ref, bp_ref, nrm_ref, wd1_ref, bd1_ref, wd2_ref, bd2_ref,
              wd3_ref, bd3_ref, cand_ref):
    fb = f_ref[0].T
    xyzb = xyz_ref[...].T
    x = jnp.concatenate([fb, xyzb], axis=-1)
    h = lax.dot(x, w1_ref[...]) + b1_ref[...]
    h = jnp.where(h >= 0, h, 0.01 * h)
    h = lax.dot(h, w2_ref[...]) + b2_ref[...]
    h = jnp.where(h >= 0, h, 0.01 * h)
    h = lax.dot(h, w3_ref[...]) + b3_ref[...]
    h = jnp.maximum(h, 0.0)
    p = (lax.dot(h, wp_ref[...]) + bp_ref[...]) / nrm_ref[0, 0]
    y = jax.nn.sigmoid(p)
    px = fb * y
    d = lax.dot(px, wd1_ref[...]) + bd1_ref[...]
    d = jnp.maximum(d, 0.0)
    d = lax.dot(d, wd2_ref[...]) + bd2_ref[...]
    d = jnp.maximum(d, 0.0)
    d = lax.dot(d, wd3_ref[...]) + bd3_ref[...]
    cand = d + jnp.concatenate(
        [xyzb, p, jnp.zeros((xyzb.shape[0], 124), jnp.float32)], axis=-1)
    cand_ref[...] = cand


def _full(shape):
    return pl.BlockSpec(shape, lambda *_: tuple(0 for _ in shape))


_NW = 32
_G_CHUNK = 928
_G_NCHUNK = 16
_G_PER_W = _G_CHUNK * _G_NCHUNK
_G_TOTAL = _G_PER_W * _NW


def _sc_gather_body(cand_hbm, idx_hbm, out_hbm, idx_v, rows_v, sem):
    wid = lax.axis_index("s") * 2 + lax.axis_index("c")
    base = wid * _G_PER_W
    for c in range(_G_NCHUNK):
        off = base + c * _G_CHUNK
        pltpu.sync_copy(idx_hbm.at[pl.ds(off, _G_CHUNK)], idx_v)
        pltpu.async_copy(cand_hbm.at[idx_v], rows_v, sem).wait()
        pltpu.sync_copy(rows_v, out_hbm.at[pl.ds(off, _G_CHUNK)])


@functools.lru_cache(maxsize=1)
def _make_sc_gather():
    return pl.kernel(
        _sc_gather_body,
        out_type=jax.ShapeDtypeStruct((_G_TOTAL, 128), jnp.float32),
        mesh=plsc.VectorSubcoreMesh(core_axis_name="c", subcore_axis_name="s"),
        scratch_types=[
            pltpu.VMEM((_G_CHUNK,), jnp.int32),
            pltpu.VMEM((_G_CHUNK, 128), jnp.float32),
            pltpu.SemaphoreType.DMA,
        ],
        compiler_params=pltpu.CompilerParams(use_tc_tiling_on_sc=True),
    )


def _sc_gather(cand, gidx):
    return _make_sc_gather()(cand, gidx)


@jax.jit
def kernel(xyz, f, W1, b1, W2, b2, W3, b3, Wp, bp, Wd1, bd1, Wd2, bd2, Wd3, bd3):
    BN = _B * _N
    num_out = int(_N * _PERCENT)
    f_t = jnp.transpose(f, (0, 2, 1))
    xyz_t = jnp.transpose(xyz, (2, 0, 1)).reshape(_PC, BN)
    nrm = jnp.linalg.norm(Wp).reshape(1, 1)
    wd3t = jnp.zeros((_IN // 4, 128), jnp.float32).at[:, :_PC].set(Wd3.T)
    bd3p = jnp.zeros((1, 128), jnp.float32).at[0, :_PC].set(bd3)

    nb = _N // _TN
    grid = (_B, nb)
    cand = pl.pallas_call(
        _mlp_body,
        grid=grid,
        in_specs=[
            pl.BlockSpec((1, _IN, _TN), lambda b, i: (b, 0, i)),
            pl.BlockSpec((_PC, _TN), lambda b, i: (0, b * nb + i)),
            _full((_IN + _PC, _H)),
            _full((1, _H)),
            _full((_H, _H)),
            _full((1, _H)),
            _full((_H, 32)),
            _full((1, 32)),
            _full((32, 1)),
            _full((1, 1)),
            _full((1, 1)),
            _full((_IN, _IN // 2)),
            _full((1, _IN // 2)),
            _full((_IN // 2, _IN // 4)),
            _full((1, _IN // 4)),
            _full((_IN // 4, 128)),
            _full((1, 128)),
        ],
        out_specs=pl.BlockSpec((_TN, 128), lambda b, i: (b * nb + i, 0)),
        out_shape=jax.ShapeDtypeStruct((BN, 128), jnp.float32),
        compiler_params=pltpu.CompilerParams(
            dimension_semantics=("arbitrary", "arbitrary"),
        ),
    )(f_t, xyz_t, W1.T, b1.reshape(1, _H), W2.T, b2.reshape(1, _H), W3.T,
      b3.reshape(1, 32), Wp.T, bp.reshape(1, 1), nrm, Wd1.T,
      bd1.reshape(1, _IN // 2), Wd2.T, bd2.reshape(1, _IN // 4), wd3t, bd3p)

    probs = cand[:, 3].reshape(_B, _N)
    prob_idx = jnp.argsort(-probs, axis=-1)
    clean_idx = prob_idx[:, num_out:]
    nc = _N - num_out
    gidx = (jnp.arange(_B, dtype=jnp.int32)[:, None] * _N + clean_idx).reshape(-1)
    gidx = jnp.pad(gidx, (0, _G_TOTAL - _B * nc))
    gathered = _sc_gather(cand, gidx)
    est_xyz = gathered[: _B * nc, :_PC].reshape(_B, nc, _PC)
    return (prob_idx, est_xyz)
